# baseline (speedup 1.0000x reference)
import jax
import jax.numpy as jnp
from jax.experimental import pallas as pl
from jax.experimental.pallas import tpu as pltpu

K = 32
BR = 128
CH = 128


def _key_to_float(key_u32):
    sign = jnp.uint32(0x80000000)
    u = jnp.where(key_u32 >= sign, key_u32 ^ sign, ~key_u32)
    return jax.lax.bitcast_convert_type(u, jnp.float32)


def _topk_mask_kernel(x_ref, w_ref, b_ref, o_ref, khi_ref, klo_ref, kc_ref):
    i16_1 = jnp.int16(1)
    i16_0 = jnp.int16(0)
    rows = o_ref.shape[0]
    d = o_ref.shape[1]

    z = jax.lax.dot_general(
        x_ref[...], w_ref[...],
        dimension_numbers=(((1,), (0,)), ((), ())),
        preferred_element_type=jnp.float32,
    ) + b_ref[...]
    o_ref[...] = z

    zu = jax.lax.bitcast_convert_type(z, jnp.uint32)
    sign = jnp.uint32(0x80000000)
    ku = jnp.where(zu < sign, zu | sign, ~zu)
    kub = ku ^ jnp.uint32(0x80008000)
    khi_ref[...] = jax.lax.bitcast_convert_type(
        (kub >> 16).astype(jnp.uint16), jnp.int16)
    klo_ref[...] = jax.lax.bitcast_convert_type(
        kub.astype(jnp.uint16), jnp.int16)

    def to_s16(cand_u32):
        return jax.lax.bitcast_convert_type(
            (cand_u32 ^ jnp.uint32(0x8000)).astype(jnp.uint16), jnp.int16)

    def count_ge(ref, cand_s16):
        acc = jnp.zeros((rows, CH), jnp.int16)
        for j in range(0, d, CH):
            acc = acc + jnp.where(ref[:, j:j + CH] >= cand_s16, i16_1, i16_0)
        return jnp.sum(acc.astype(jnp.int32), axis=1, keepdims=True)

    def step_hi(i, t_hi):
        bit = jax.lax.shift_left(jnp.uint32(1), (15 - i).astype(jnp.uint32))
        cand = t_hi | bit
        cnt = count_ge(khi_ref, to_s16(cand))
        return jnp.where(cnt >= K, cand, t_hi)

    t_hi32 = jax.lax.fori_loop(0, 16, step_hi,
                               jnp.zeros((rows, 1), jnp.uint32))
    t_hi = to_s16(t_hi32)

    khi_all = khi_ref[...]
    kc_ref[...] = jnp.where(
        khi_all == t_hi, klo_ref[...],
        jnp.where(khi_all > t_hi, jnp.int16(32767), jnp.int16(-32768)))

    def step_lo(i, t_lo):
        bit = jax.lax.shift_left(jnp.uint32(1), (15 - i).astype(jnp.uint32))
        cand = t_lo | bit
        cnt = count_ge(kc_ref, to_s16(cand))
        return jnp.where(cnt >= K, cand, t_lo)

    t_lo32 = jax.lax.fori_loop(0, 16, step_lo,
                               jnp.zeros((rows, 1), jnp.uint32))

    t_key = jax.lax.shift_left(t_hi32, jnp.uint32(16)) | t_lo32
    thresh = _key_to_float(t_key)
    zz = o_ref[...]
    o_ref[...] = jnp.where(zz >= thresh, zz, 0.0)


def kernel(x, W_enc, b_enc):
    n_tok, d_in = x.shape
    d_dict = W_enc.shape[0]
    wt = W_enc.T.astype(jnp.bfloat16)
    xb = x.astype(jnp.bfloat16)
    b2 = b_enc.reshape(1, d_dict)
    return pl.pallas_call(
        _topk_mask_kernel,
        grid=(n_tok // BR,),
        in_specs=[
            pl.BlockSpec((BR, d_in), lambda i: (i, 0)),
            pl.BlockSpec((d_in, d_dict), lambda i: (0, 0)),
            pl.BlockSpec((1, d_dict), lambda i: (0, 0)),
        ],
        out_specs=pl.BlockSpec((BR, d_dict), lambda i: (i, 0)),
        out_shape=jax.ShapeDtypeStruct((n_tok, d_dict), jnp.float32),
        scratch_shapes=[
            pltpu.VMEM((BR, d_dict), jnp.int16),
            pltpu.VMEM((BR, d_dict), jnp.int16),
            pltpu.VMEM((BR, d_dict), jnp.int16),
        ],
    )(xb, wt, b2)
